# baseline (speedup 1.0000x reference)
import functools

import jax
import jax.numpy as jnp
from jax import lax
from jax.experimental import pallas as pl
from jax.experimental.pallas import tpu as pltpu
from jax.experimental.pallas import tpu_sc as plsc

VOCAB = 1000000
DIM = 64
BATCH = 4096
SEQ = 200

NUM_CORES = 2
NUM_SUBCORES = 16
NW = NUM_CORES * NUM_SUBCORES

NVB = VOCAB // 128
VTAIL = VOCAB - NVB * 128
FULL_PER_W = NVB // NW
VB_REM = NVB - FULL_PER_W * NW
BLK = 128 * DIM

_mesh = plsc.VectorSubcoreMesh(core_axis_name="c", subcore_axis_name="s")


@functools.partial(
    pl.kernel,
    mesh=_mesh,
    out_type=jax.ShapeDtypeStruct((VOCAB * DIM,), jnp.float32),
    compiler_params=pltpu.CompilerParams(use_tc_tiling_on_sc=True,
                                         needs_layout_passes=False),
    scratch_types=[
        pltpu.VMEM((DIM, 128), jnp.float32),
        pltpu.VMEM((DIM, 128), jnp.float32),
        pltpu.VMEM((BLK,), jnp.float32),
        pltpu.VMEM((BLK,), jnp.float32),
        pltpu.VMEM((DIM, VTAIL), jnp.float32),
        pltpu.SemaphoreType.DMA,
        pltpu.SemaphoreType.DMA,
        pltpu.SemaphoreType.DMA,
        pltpu.SemaphoreType.DMA,
    ],
)
def _transpose_kernel(tt_hbm, tail_hbm, out_hbm, in0, in1, ob0, ob1, tin,
                      si0, si1, so0, so1):
    wid = lax.axis_index("s") * NUM_CORES + lax.axis_index("c")
    inb = (in0, in1)
    outb = (ob0, ob1)
    si = (si0, si1)
    so = (so0, so1)

    iota = lax.iota(jnp.int32, 16)
    dvecs = [iota + (16 * t) for t in range(4)]

    def start_in(vb, b):
        pltpu.async_copy(tt_hbm.at[:, pl.ds(vb * 128, 128)], inb[b], si[b])

    def wait_in(b):
        pltpu.make_async_copy(tt_hbm.at[:, pl.ds(0, 128)], inb[b],
                              si[b]).wait()

    def start_out(vb, b):
        pltpu.async_copy(outb[b], out_hbm.at[pl.ds(vb * BLK, BLK)], so[b])

    def wait_out(b):
        pltpu.make_async_copy(outb[b], out_hbm.at[pl.ds(0, BLK)],
                              so[b]).wait()

    def transpose_block(b):
        def tbody(c, carry):
            vbase = jnp.full((16,), 0, jnp.int32) + c * 16
            for vpo in range(16):
                vsplat = vbase + vpo
                for t in range(4):
                    vec = plsc.load_gather(inb[b], [dvecs[t], vsplat])
                    outb[b][pl.ds(c * (16 * DIM) + vpo * DIM + 16 * t,
                                  16)] = vec
            return carry

        lax.fori_loop(0, 8, tbody, 0)

    def step(vb, b, first):
        wait_in(b)
        if not first:
            wait_out(b)
        transpose_block(b)
        start_out(vb, b)

    vb0 = wid
    start_in(vb0, 0)
    start_in(vb0 + NW, 1)
    step(vb0, 0, True)
    start_in(vb0 + 2 * NW, 0)
    step(vb0 + NW, 1, True)

    def body(i, carry):
        vb = vb0 + (2 * i + 2) * NW
        start_in(vb + NW, 1)
        step(vb, 0, False)
        nxt = jnp.minimum(vb + 2 * NW, NVB - 1)
        start_in(nxt, 0)
        step(vb + NW, 1, False)
        return carry

    lax.fori_loop(0, (FULL_PER_W - 2) // 2, body, 0)
    wait_in(0)

    @pl.when(wid < VB_REM)
    def _():
        wait_out(0)
        transpose_block(0)
        start_out(vb0 + FULL_PER_W * NW, 0)

    @pl.when(wid == VB_REM)
    def _():
        pltpu.sync_copy(tail_hbm, tin)
        wait_out(1)

        def tail_body(c, carry):
            vbase = jnp.full((16,), 0, jnp.int32) + c * 16
            for vpo in range(16):
                vsplat = vbase + vpo
                for t in range(4):
                    vec = plsc.load_gather(tin, [dvecs[t], vsplat])
                    ob1[pl.ds(c * (16 * DIM) + vpo * DIM + 16 * t,
                              16)] = vec
            return carry

        lax.fori_loop(0, VTAIL // 16, tail_body, 0)
        pltpu.async_copy(ob1.at[pl.ds(0, VTAIL * DIM)],
                         out_hbm.at[pl.ds(NVB * BLK, VTAIL * DIM)], so1)
        pltpu.make_async_copy(ob1.at[pl.ds(0, VTAIL * DIM)],
                              out_hbm.at[pl.ds(0, VTAIL * DIM)], so1).wait()

    @pl.when(wid != VB_REM)
    def _():
        wait_out(1)

    wait_out(0)


ROWS_W = BATCH // NW


@functools.partial(
    pl.kernel,
    mesh=_mesh,
    out_type=jax.ShapeDtypeStruct((SEQ, DIM // 8, NW, 1024), jnp.float32),
    compiler_params=pltpu.CompilerParams(use_tc_tiling_on_sc=False,
                                         needs_layout_passes=False),
    scratch_types=[
        pltpu.VMEM((SEQ, ROWS_W), jnp.int32),
        pltpu.VMEM((ROWS_W, DIM), jnp.float32),
        pltpu.VMEM((ROWS_W, DIM), jnp.float32),
        pltpu.VMEM((DIM // 8, 1024), jnp.float32),
        pltpu.VMEM((DIM // 8, 1024), jnp.float32),
        pltpu.SemaphoreType.DMA,
        pltpu.SemaphoreType.DMA,
        pltpu.SemaphoreType.DMA,
        pltpu.SemaphoreType.DMA,
    ],
)
def _gather_kernel(idxt_hbm, table_hbm, out_hbm, idx_v, g0, g1, t0, t1,
                   sg0, sg1, so0, so1):
    wid = lax.axis_index("s") * NUM_CORES + lax.axis_index("c")

    gb = (g0, g1)
    tb = (t0, t1)
    sg = (sg0, sg1)
    so = (so0, so1)

    pltpu.sync_copy(idxt_hbm.at[:, pl.ds(wid * ROWS_W, ROWS_W)], idx_v)

    iota = lax.iota(jnp.int32, 16)
    bvecs = [iota + (16 * k) for k in range(8)]

    def start_gather(s, b):
        pltpu.async_copy(table_hbm.at[idx_v.at[s]], gb[b], sg[b])

    def wait_gather(b):
        pltpu.make_async_copy(table_hbm.at[idx_v.at[0]], gb[b], sg[b]).wait()

    def transpose_block(b):
        def tbody(c, carry):
            dbase = jnp.full((16,), 0, jnp.int32) + c * 8
            for do in range(8):
                dsplat = dbase + do
                for k in range(8):
                    vec = plsc.load_gather(gb[b], [bvecs[k], dsplat])
                    tb[b][c, pl.ds(do * 128 + 16 * k, 16)] = vec
            return carry

        lax.fori_loop(0, DIM // 8, tbody, 0)

    def start_out(s, b):
        pltpu.async_copy(tb[b], out_hbm.at[s, :, wid], so[b])

    def wait_out(b):
        pltpu.make_async_copy(tb[b], out_hbm.at[0, :, wid], so[b]).wait()

    def step(s, b, first):
        wait_gather(b)
        if not first:
            wait_out(b)
        transpose_block(b)
        start_out(s, b)

    start_gather(0, 0)
    start_gather(1, 1)
    step(0, 0, True)
    start_gather(2, 0)
    step(1, 1, True)

    def body(j, carry):
        s = 2 * j + 2
        start_gather(s + 1, 1)
        step(s, 0, False)
        start_gather(jnp.minimum(s + 2, SEQ - 1), 0)
        step(s + 1, 1, False)
        return carry

    lax.fori_loop(0, (SEQ - 2) // 2, body, 0)
    wait_gather(0)
    wait_out(0)
    wait_out(1)


def kernel(input_ids, table):
    tt = table.T
    tail = lax.slice(tt, (0, NVB * 128), (DIM, VOCAB))
    tlin = _transpose_kernel(tt, tail)
    t2d = tlin.reshape(VOCAB, DIM)
    idxt = input_ids.astype(jnp.int32).T
    out5 = _gather_kernel(idxt, t2d)
    out5 = out5.reshape(SEQ, DIM // 8, NW, 8, 128)
    return out5.transpose(2, 4, 0, 1, 3).reshape(BATCH, SEQ, DIM)


# speedup vs baseline: 1.9088x; 1.9088x over previous
import functools

import jax
import jax.numpy as jnp
from jax import lax
from jax.experimental import pallas as pl
from jax.experimental.pallas import tpu as pltpu
from jax.experimental.pallas import tpu_sc as plsc

VOCAB = 1000000
DIM = 64
BATCH = 4096
SEQ = 200

NUM_CORES = 2
NUM_SUBCORES = 16
NW = NUM_CORES * NUM_SUBCORES

NVB = VOCAB // 128
VTAIL = VOCAB - NVB * 128
FULL_PER_W = NVB // NW
VB_REM = NVB - FULL_PER_W * NW
BLK = 128 * DIM

_mesh = plsc.VectorSubcoreMesh(core_axis_name="c", subcore_axis_name="s")


@functools.partial(
    pl.kernel,
    mesh=_mesh,
    out_type=jax.ShapeDtypeStruct((VOCAB * DIM,), jnp.float32),
    compiler_params=pltpu.CompilerParams(use_tc_tiling_on_sc=True,
                                         needs_layout_passes=False),
    scratch_types=[
        pltpu.VMEM((DIM, 128), jnp.float32),
        pltpu.VMEM((DIM, 128), jnp.float32),
        pltpu.VMEM((BLK,), jnp.float32),
        pltpu.VMEM((BLK,), jnp.float32),
        pltpu.VMEM((DIM, VTAIL), jnp.float32),
        pltpu.SemaphoreType.DMA,
        pltpu.SemaphoreType.DMA,
        pltpu.SemaphoreType.DMA,
        pltpu.SemaphoreType.DMA,
    ],
)
def _transpose_kernel(tt_hbm, tail_hbm, out_hbm, in0, in1, ob0, ob1, tin,
                      si0, si1, so0, so1):
    wid = lax.axis_index("s") * NUM_CORES + lax.axis_index("c")
    inb = (in0, in1)
    outb = (ob0, ob1)
    si = (si0, si1)
    so = (so0, so1)

    iota = lax.iota(jnp.int32, 16)
    dvecs = [iota + (16 * t) for t in range(4)]

    def start_in(vb, b):
        pltpu.async_copy(tt_hbm.at[:, pl.ds(vb * 128, 128)], inb[b], si[b])

    def wait_in(b):
        pltpu.make_async_copy(tt_hbm.at[:, pl.ds(0, 128)], inb[b],
                              si[b]).wait()

    def start_out(vb, b):
        pltpu.async_copy(outb[b], out_hbm.at[pl.ds(vb * BLK, BLK)], so[b])

    def wait_out(b):
        pltpu.make_async_copy(outb[b], out_hbm.at[pl.ds(0, BLK)],
                              so[b]).wait()

    def transpose_block(b):
        @plsc.parallel_loop(0, 128, unroll=4)
        def _(vp):
            vsplat = jnp.full((16,), 0, jnp.int32) + vp
            for t in range(4):
                vec = plsc.load_gather(inb[b], [dvecs[t], vsplat])
                outb[b][pl.ds(vp * DIM + 16 * t, 16)] = vec

    def step(vb, b, first):
        wait_in(b)
        if not first:
            wait_out(b)
        transpose_block(b)
        start_out(vb, b)

    vb0 = wid
    start_in(vb0, 0)
    start_in(vb0 + NW, 1)
    step(vb0, 0, True)
    start_in(vb0 + 2 * NW, 0)
    step(vb0 + NW, 1, True)

    def body(i, carry):
        vb = vb0 + (2 * i + 2) * NW
        start_in(vb + NW, 1)
        step(vb, 0, False)
        nxt = jnp.minimum(vb + 2 * NW, NVB - 1)
        start_in(nxt, 0)
        step(vb + NW, 1, False)
        return carry

    lax.fori_loop(0, (FULL_PER_W - 2) // 2, body, 0)
    wait_in(0)

    @pl.when(wid < VB_REM)
    def _():
        wait_out(0)
        transpose_block(0)
        start_out(vb0 + FULL_PER_W * NW, 0)

    @pl.when(wid == VB_REM)
    def _():
        pltpu.sync_copy(tail_hbm, tin)
        wait_out(1)

        @plsc.parallel_loop(0, VTAIL, unroll=4)
        def _(vp):
            vsplat = jnp.full((16,), 0, jnp.int32) + vp
            for t in range(4):
                vec = plsc.load_gather(tin, [dvecs[t], vsplat])
                ob1[pl.ds(vp * DIM + 16 * t, 16)] = vec
        pltpu.async_copy(ob1.at[pl.ds(0, VTAIL * DIM)],
                         out_hbm.at[pl.ds(NVB * BLK, VTAIL * DIM)], so1)
        pltpu.make_async_copy(ob1.at[pl.ds(0, VTAIL * DIM)],
                              out_hbm.at[pl.ds(0, VTAIL * DIM)], so1).wait()

    @pl.when(wid != VB_REM)
    def _():
        wait_out(1)

    wait_out(0)


ROWS_W = BATCH // NW


@functools.partial(
    pl.kernel,
    mesh=_mesh,
    out_type=jax.ShapeDtypeStruct((SEQ, DIM // 8, NW, 1024), jnp.float32),
    compiler_params=pltpu.CompilerParams(use_tc_tiling_on_sc=False,
                                         needs_layout_passes=False),
    scratch_types=[
        pltpu.VMEM((SEQ, ROWS_W), jnp.int32),
        pltpu.VMEM((ROWS_W, DIM), jnp.float32),
        pltpu.VMEM((ROWS_W, DIM), jnp.float32),
        pltpu.VMEM((DIM // 8, 1024), jnp.float32),
        pltpu.VMEM((DIM // 8, 1024), jnp.float32),
        pltpu.SemaphoreType.DMA,
        pltpu.SemaphoreType.DMA,
        pltpu.SemaphoreType.DMA,
        pltpu.SemaphoreType.DMA,
    ],
)
def _gather_kernel(idxt_hbm, table_hbm, out_hbm, idx_v, g0, g1, t0, t1,
                   sg0, sg1, so0, so1):
    wid = lax.axis_index("s") * NUM_CORES + lax.axis_index("c")

    gb = (g0, g1)
    tb = (t0, t1)
    sg = (sg0, sg1)
    so = (so0, so1)

    pltpu.sync_copy(idxt_hbm.at[:, pl.ds(wid * ROWS_W, ROWS_W)], idx_v)

    iota = lax.iota(jnp.int32, 16)
    bvecs = [iota + (16 * k) for k in range(8)]

    def start_gather(s, b):
        pltpu.async_copy(table_hbm.at[idx_v.at[s]], gb[b], sg[b])

    def wait_gather(b):
        pltpu.make_async_copy(table_hbm.at[idx_v.at[0]], gb[b], sg[b]).wait()

    def transpose_block(b):
        @plsc.parallel_loop(0, DIM, unroll=4)
        def _(d):
            dsplat = jnp.full((16,), 0, jnp.int32) + d
            r = d // 8
            off = (d % 8) * 128
            for k in range(8):
                vec = plsc.load_gather(gb[b], [bvecs[k], dsplat])
                tb[b][r, pl.ds(off + 16 * k, 16)] = vec

    def start_out(s, b):
        pltpu.async_copy(tb[b], out_hbm.at[s, :, wid], so[b])

    def wait_out(b):
        pltpu.make_async_copy(tb[b], out_hbm.at[0, :, wid], so[b]).wait()

    def step(s, b, first):
        wait_gather(b)
        if not first:
            wait_out(b)
        transpose_block(b)
        start_out(s, b)

    start_gather(0, 0)
    start_gather(1, 1)
    step(0, 0, True)
    start_gather(2, 0)
    step(1, 1, True)

    def body(j, carry):
        s = 2 * j + 2
        start_gather(s + 1, 1)
        step(s, 0, False)
        start_gather(jnp.minimum(s + 2, SEQ - 1), 0)
        step(s + 1, 1, False)
        return carry

    lax.fori_loop(0, (SEQ - 2) // 2, body, 0)
    wait_gather(0)
    wait_out(0)
    wait_out(1)


def kernel(input_ids, table):
    tt = table.T
    tail = lax.slice(tt, (0, NVB * 128), (DIM, VOCAB))
    tlin = _transpose_kernel(tt, tail)
    t2d = tlin.reshape(VOCAB, DIM)
    idxt = input_ids.astype(jnp.int32).T
    out5 = _gather_kernel(idxt, t2d)
    out5 = out5.reshape(SEQ, DIM // 8, NW, 8, 128)
    return out5.transpose(2, 4, 0, 1, 3).reshape(BATCH, SEQ, DIM)
